# Initial kernel scaffold; baseline (speedup 1.0000x reference)
import jax
import jax.numpy as jnp
from jax.experimental import pallas as pl


def kernel(x, table):
    raise NotImplementedError("write your pallas kernel here")


# speedup vs baseline: 2.4307x; 2.4307x over previous
import functools

import jax
import jax.numpy as jnp
from jax import lax
from jax.experimental import pallas as pl
from jax.experimental.pallas import tpu as pltpu
from jax.experimental.pallas import tpu_sc as plsc

L = 200
D = 64
LANES = 16


def _pe_table():
    position = jnp.arange(L, dtype=jnp.float32)[:, None]
    div_term = jnp.exp(
        jnp.arange(0, D, 2, dtype=jnp.float32) * (-jnp.log(10000.0) / D)
    )
    pe = jnp.zeros((L, D), dtype=jnp.float32)
    pe = pe.at[:, 0::2].set(jnp.sin(position * div_term))
    pe = pe.at[:, 1::2].set(jnp.cos(position * div_term))
    return pe


@functools.partial(jax.jit, static_argnames=("batch", "vocab"))
def _run(x_flat, table, pe, *, batch, vocab):
    info = plsc.get_sparse_core_info()
    nc, ns = info.num_cores, info.num_subcores
    nw = nc * ns
    n = batch * L
    per_w = n // nw
    seq_chunk = 2
    c_rows = seq_chunk * L
    n_chunks = per_w // c_rows

    mesh = plsc.VectorSubcoreMesh(core_axis_name="c", subcore_axis_name="s")

    @functools.partial(
        pl.kernel,
        out_type=jax.ShapeDtypeStruct((n, D), jnp.float32),
        mesh=mesh,
        scratch_types=[
            pltpu.VMEM((c_rows,), jnp.int32),
            pltpu.VMEM((c_rows, D), jnp.float32),
            pltpu.VMEM((L, D), jnp.float32),
            pltpu.SemaphoreType.DMA,
        ],
        compiler_params=pltpu.CompilerParams(use_tc_tiling_on_sc=False),
    )
    def k(x_hbm, table_hbm, pe_hbm, out_hbm, idx_v, rows_v, pe_v, sem):
        wid = lax.axis_index("s") * nc + lax.axis_index("c")
        pltpu.sync_copy(pe_hbm, pe_v)
        w_base = wid * per_w

        @pl.loop(0, n_chunks)
        def _chunk(ci):
            base = w_base + ci * c_rows
            pltpu.sync_copy(x_hbm.at[pl.ds(base, c_rows)], idx_v)
            pltpu.async_copy(table_hbm.at[idx_v], rows_v, sem).wait()

            @pl.loop(0, L, unroll=4)
            def _add(li):
                for s in range(seq_chunk):
                    for c4 in range(D // LANES):
                        sl = pl.ds(c4 * LANES, LANES)
                        rows_v[s * L + li, sl] = rows_v[s * L + li, sl] + pe_v[li, sl]

            pltpu.sync_copy(rows_v, out_hbm.at[pl.ds(base, c_rows)])

    return k(x_flat, table, pe)


def kernel(x, table):
    batch, seq = x.shape
    vocab = table.shape[0]
    x_flat = x.reshape(-1).astype(jnp.int32)
    pe = _pe_table()
    out = _run(x_flat, table, pe, batch=batch, vocab=vocab)
    return out.reshape(batch, seq, D)


# speedup vs baseline: 2.7589x; 1.1350x over previous
import functools

import jax
import jax.numpy as jnp
from jax import lax
from jax.experimental import pallas as pl
from jax.experimental.pallas import tpu as pltpu
from jax.experimental.pallas import tpu_sc as plsc

L = 200
D = 64
LANES = 16


def _pe_table():
    position = jnp.arange(L, dtype=jnp.float32)[:, None]
    div_term = jnp.exp(
        jnp.arange(0, D, 2, dtype=jnp.float32) * (-jnp.log(10000.0) / D)
    )
    pe = jnp.zeros((L, D), dtype=jnp.float32)
    pe = pe.at[:, 0::2].set(jnp.sin(position * div_term))
    pe = pe.at[:, 1::2].set(jnp.cos(position * div_term))
    return pe


@functools.partial(jax.jit, static_argnames=("batch", "vocab"))
def _run(x, table, pe, *, batch, vocab):
    info = plsc.get_sparse_core_info()
    nc, ns = info.num_cores, info.num_subcores
    nw = nc * ns
    seqs_per_w = batch // nw
    n_chunks = seqs_per_w

    mesh = plsc.VectorSubcoreMesh(core_axis_name="c", subcore_axis_name="s")

    @functools.partial(
        pl.kernel,
        out_type=jax.ShapeDtypeStruct((batch, L, D), jnp.float32),
        mesh=mesh,
        scratch_types=[
            pltpu.VMEM((seqs_per_w, L), jnp.int32),
            pltpu.VMEM((L, D), jnp.float32),
            pltpu.VMEM((L, D), jnp.float32),
            pltpu.VMEM((L, D), jnp.float32),
            pltpu.VMEM((L, D), jnp.float32),
            pltpu.VMEM((L, D), jnp.float32),
            pltpu.SemaphoreType.DMA,
            pltpu.SemaphoreType.DMA,
            pltpu.SemaphoreType.DMA,
            pltpu.SemaphoreType.DMA,
        ],
        compiler_params=pltpu.CompilerParams(use_tc_tiling_on_sc=False),
    )
    def k(x_hbm, table_hbm, pe_hbm, out_hbm,
          idx_all, pe_v, g0, g1, w0, w1, gs0, gs1, ws0, ws1):
        gbuf = (g0, g1)
        wbuf = (w0, w1)
        gsem = (gs0, gs1)
        wsem = (ws0, ws1)
        wid = lax.axis_index("s") * nc + lax.axis_index("c")
        seq_base = wid * seqs_per_w
        pltpu.sync_copy(pe_hbm, pe_v)
        pltpu.sync_copy(x_hbm.at[pl.ds(seq_base, seqs_per_w)], idx_all)

        def start_gather(ci, b):
            pltpu.async_copy(table_hbm.at[idx_all.at[ci]], gbuf[b], gsem[b])

        def wait_gather(ci, b):
            pltpu.make_async_copy(
                table_hbm.at[idx_all.at[ci]], gbuf[b], gsem[b]
            ).wait()

        def start_write(ci, b):
            pltpu.async_copy(wbuf[b], out_hbm.at[seq_base + ci], wsem[b])

        def wait_write(ci, b):
            pltpu.make_async_copy(
                wbuf[b], out_hbm.at[seq_base + ci], wsem[b]
            ).wait()

        for b in range(2):
            start_gather(b, b)

        @pl.loop(0, n_chunks, step=2)
        def _chunks(g):
            for b in range(2):
                ci = g + b
                wait_gather(ci, b)

                @pl.when(ci >= 2)
                def _():
                    wait_write(ci - 2, b)

                @pl.loop(0, L, unroll=8)
                def _add(li):
                    for c4 in range(D // LANES):
                        sl = pl.ds(c4 * LANES, LANES)
                        wbuf[b][li, sl] = gbuf[b][li, sl] + pe_v[li, sl]

                @pl.when(ci + 2 < n_chunks)
                def _():
                    start_gather(ci + 2, b)

                start_write(ci, b)

        for b in range(2):
            wait_write(n_chunks - 2 + b, b)

    return k(x, table, pe)


def kernel(x, table):
    batch, seq = x.shape
    vocab = table.shape[0]
    pe = _pe_table()
    out = _run(x.astype(jnp.int32), table, pe, batch=batch, vocab=vocab)
    return out


# speedup vs baseline: 4.6492x; 1.6852x over previous
import functools

import jax
import jax.numpy as jnp
from jax import lax
from jax.experimental import pallas as pl
from jax.experimental.pallas import tpu as pltpu
from jax.experimental.pallas import tpu_sc as plsc

L = 200
D = 64
LANES = 16
BT = 128
TPAD = 129


def _pe_table():
    position = jnp.arange(L, dtype=jnp.float32)[:, None]
    div_term = jnp.exp(
        jnp.arange(0, D, 2, dtype=jnp.float32) * (-jnp.log(10000.0) / D)
    )
    pe = jnp.zeros((L, D), dtype=jnp.float32)
    pe = pe.at[:, 0::2].set(jnp.sin(position * div_term))
    pe = pe.at[:, 1::2].set(jnp.cos(position * div_term))
    return pe


@functools.partial(jax.jit, static_argnames=("batch", "vocab"))
def _run(xq, table, pe, *, batch, vocab):
    info = plsc.get_sparse_core_info()
    nc, ns = info.num_cores, info.num_subcores
    nw = nc * ns
    nbt = batch // BT
    n_blocks = L * nbt
    blocks_per_w = n_blocks // nw

    mesh = plsc.VectorSubcoreMesh(core_axis_name="c", subcore_axis_name="s")

    @functools.partial(
        pl.kernel,
        out_type=jax.ShapeDtypeStruct((L, D // 8, nbt, 8, BT), jnp.float32),
        mesh=mesh,
        scratch_types=[
            pltpu.VMEM((blocks_per_w, BT), jnp.int32),
            pltpu.VMEM((L, D), jnp.float32),
            pltpu.VMEM((BT, D), jnp.float32),
            pltpu.VMEM((BT, D), jnp.float32),
            pltpu.VMEM((D, TPAD), jnp.float32),
            pltpu.VMEM((D, TPAD), jnp.float32),
            pltpu.SemaphoreType.DMA,
            pltpu.SemaphoreType.DMA,
            pltpu.SemaphoreType.DMA,
            pltpu.SemaphoreType.DMA,
        ],
        compiler_params=pltpu.CompilerParams(
            use_tc_tiling_on_sc=False, needs_layout_passes=False
        ),
    )
    def k(x_hbm, table_hbm, pe_hbm, out_hbm,
          idx_all, pe_v, g0, g1, t0, t1, gs0, gs1, ws0, ws1):
        gbuf = (g0, g1)
        tbuf = (t0, t1)
        gsem = (gs0, gs1)
        wsem = (ws0, ws1)
        wid = lax.axis_index("s") * nc + lax.axis_index("c")
        blk_base = wid * blocks_per_w
        pltpu.sync_copy(pe_hbm, pe_v)
        pltpu.sync_copy(x_hbm.at[pl.ds(blk_base, blocks_per_w)], idx_all)

        iota = lax.iota(jnp.int32, LANES)
        rows = [iota + (g * LANES) for g in range(D // LANES)]

        def start_gather(kk, b):
            pltpu.async_copy(table_hbm.at[idx_all.at[kk]], gbuf[b], gsem[b])

        def wait_gather(kk, b):
            pltpu.make_async_copy(
                table_hbm.at[idx_all.at[kk]], gbuf[b], gsem[b]
            ).wait()

        def lc_of(kk):
            bid = blk_base + kk
            return bid // nbt, bid % nbt

        def start_writes(kk, b):
            l, tc = lc_of(kk)
            for tr in range(D // 8):
                pltpu.async_copy(
                    tbuf[b].at[pl.ds(tr * 8, 8), pl.ds(0, BT)],
                    out_hbm.at[l, tr, tc],
                    wsem[b],
                )

        def wait_writes(kk, b):
            l, tc = lc_of(kk)
            for tr in range(D // 8):
                pltpu.make_async_copy(
                    tbuf[b].at[pl.ds(tr * 8, 8), pl.ds(0, BT)],
                    out_hbm.at[l, tr, tc],
                    wsem[b],
                ).wait()

        for b in range(2):
            start_gather(b, b)

        @pl.loop(0, blocks_per_w, step=2)
        def _blocks(k2):
            for b in range(2):
                kk = k2 + b
                l, _ = lc_of(kk)
                wait_gather(kk, b)

                @pl.when(kk >= 2)
                def _():
                    wait_writes(kk - 2, b)

                @pl.loop(0, BT, unroll=2)
                def _tok(c):
                    col = jnp.broadcast_to(c, (LANES,)).astype(jnp.int32)
                    for g in range(D // LANES):
                        v = (gbuf[b][c, pl.ds(g * LANES, LANES)]
                             + pe_v[l, pl.ds(g * LANES, LANES)])
                        plsc.store_scatter(tbuf[b], [rows[g], col], v)

                @pl.when(kk + 2 < blocks_per_w)
                def _():
                    start_gather(kk + 2, b)

                start_writes(kk, b)

        for b in range(2):
            wait_writes(blocks_per_w - 2 + b, b)

    return k(xq, table, pe)


def kernel(x, table):
    batch, seq = x.shape
    vocab = table.shape[0]
    xq = jnp.transpose(x.astype(jnp.int32)).reshape(seq * (batch // BT), BT)
    pe = _pe_table()
    out5 = _run(xq, table, pe, batch=batch, vocab=vocab)
    return out5.transpose(2, 4, 0, 1, 3).reshape(batch, seq, D)


# speedup vs baseline: 12.0365x; 2.5890x over previous
import functools

import jax
import jax.numpy as jnp
from jax import lax
from jax.experimental import pallas as pl
from jax.experimental.pallas import tpu as pltpu
from jax.experimental.pallas import tpu_sc as plsc

L = 200
D = 64
LANES = 16
BT = 128
TPAD = 129


def _pe_table():
    position = jnp.arange(L, dtype=jnp.float32)[:, None]
    div_term = jnp.exp(
        jnp.arange(0, D, 2, dtype=jnp.float32) * (-jnp.log(10000.0) / D)
    )
    pe = jnp.zeros((L, D), dtype=jnp.float32)
    pe = pe.at[:, 0::2].set(jnp.sin(position * div_term))
    pe = pe.at[:, 1::2].set(jnp.cos(position * div_term))
    return pe


@functools.partial(jax.jit, static_argnames=("batch", "vocab"))
def _run(xq, table, pe, *, batch, vocab):
    info = plsc.get_sparse_core_info()
    nc, ns = info.num_cores, info.num_subcores
    nw = nc * ns
    nbt = batch // BT
    n_blocks = L * nbt
    blocks_per_w = n_blocks // nw

    mesh = plsc.VectorSubcoreMesh(core_axis_name="c", subcore_axis_name="s")

    @functools.partial(
        pl.kernel,
        out_type=jax.ShapeDtypeStruct((L, D // 8, nbt, 8, BT), jnp.float32),
        mesh=mesh,
        scratch_types=[
            pltpu.VMEM((blocks_per_w, BT), jnp.int32),
            pltpu.VMEM((L, D), jnp.float32),
            pltpu.VMEM((BT, D), jnp.float32),
            pltpu.VMEM((BT, D), jnp.float32),
            pltpu.VMEM((D, TPAD), jnp.float32),
            pltpu.VMEM((D, TPAD), jnp.float32),
            pltpu.SemaphoreType.DMA,
            pltpu.SemaphoreType.DMA,
            pltpu.SemaphoreType.DMA,
            pltpu.SemaphoreType.DMA,
        ],
        compiler_params=pltpu.CompilerParams(
            use_tc_tiling_on_sc=False, needs_layout_passes=False
        ),
    )
    def k(x_hbm, table_hbm, pe_hbm, out_hbm,
          idx_all, pe_v, g0, g1, t0, t1, gs0, gs1, ws0, ws1):
        gbuf = (g0, g1)
        tbuf = (t0, t1)
        gsem = (gs0, gs1)
        wsem = (ws0, ws1)
        wid = lax.axis_index("s") * nc + lax.axis_index("c")
        blk_base = wid * blocks_per_w
        pltpu.sync_copy(pe_hbm, pe_v)
        pltpu.sync_copy(x_hbm.at[pl.ds(blk_base, blocks_per_w)], idx_all)

        iota = lax.iota(jnp.int32, LANES)
        rows = [iota + (g * LANES) for g in range(D // LANES)]

        def start_gather(kk, b):
            pltpu.async_copy(table_hbm.at[idx_all.at[kk]], gbuf[b], gsem[b])

        def wait_gather(kk, b):
            pltpu.make_async_copy(
                table_hbm.at[idx_all.at[kk]], gbuf[b], gsem[b]
            ).wait()

        def lc_of(kk):
            bid = blk_base + kk
            return bid // nbt, bid % nbt

        def start_writes(kk, b):
            l, tc = lc_of(kk)
            for tr in range(D // 8):
                pltpu.async_copy(
                    tbuf[b].at[pl.ds(tr * 8, 8), pl.ds(0, BT)],
                    out_hbm.at[l, tr, tc],
                    wsem[b],
                )

        def wait_writes(kk, b):
            l, tc = lc_of(kk)
            for tr in range(D // 8):
                pltpu.make_async_copy(
                    tbuf[b].at[pl.ds(tr * 8, 8), pl.ds(0, BT)],
                    out_hbm.at[l, tr, tc],
                    wsem[b],
                ).wait()

        for b in range(2):
            start_gather(b, b)

        @pl.loop(0, blocks_per_w, step=2)
        def _blocks(k2):
            for b in range(2):
                kk = k2 + b
                l, _ = lc_of(kk)
                wait_gather(kk, b)

                @pl.when(kk >= 2)
                def _():
                    wait_writes(kk - 2, b)

                pes = [pe_v[l, pl.ds(g * LANES, LANES)]
                       for g in range(D // LANES)]

                @plsc.parallel_loop(0, BT, unroll=4)
                def _tok(c):
                    col = jnp.broadcast_to(c, (LANES,))
                    for g in range(D // LANES):
                        v = gbuf[b][c, pl.ds(g * LANES, LANES)] + pes[g]
                        plsc.store_scatter(tbuf[b], [rows[g], col], v)

                @pl.when(kk + 2 < blocks_per_w)
                def _():
                    start_gather(kk + 2, b)

                start_writes(kk, b)

        for b in range(2):
            wait_writes(blocks_per_w - 2 + b, b)

    return k(xq, table, pe)


def kernel(x, table):
    batch, seq = x.shape
    vocab = table.shape[0]
    xq = jnp.transpose(x.astype(jnp.int32)).reshape(seq * (batch // BT), BT)
    pe = _pe_table()
    out5 = _run(xq, table, pe, batch=batch, vocab=vocab)
    return out5.transpose(2, 4, 0, 1, 3).reshape(batch, seq, D)


# speedup vs baseline: 12.1068x; 1.0058x over previous
import functools

import jax
import jax.numpy as jnp
from jax import lax
from jax.experimental import pallas as pl
from jax.experimental.pallas import tpu as pltpu
from jax.experimental.pallas import tpu_sc as plsc

L = 200
D = 64
LANES = 16
BT = 128
TPAD = 129


def _pe_table():
    position = jnp.arange(L, dtype=jnp.float32)[:, None]
    div_term = jnp.exp(
        jnp.arange(0, D, 2, dtype=jnp.float32) * (-jnp.log(10000.0) / D)
    )
    pe = jnp.zeros((L, D), dtype=jnp.float32)
    pe = pe.at[:, 0::2].set(jnp.sin(position * div_term))
    pe = pe.at[:, 1::2].set(jnp.cos(position * div_term))
    return pe


@functools.partial(jax.jit, static_argnames=("batch", "vocab"))
def _run(xq, table, pe, *, batch, vocab):
    info = plsc.get_sparse_core_info()
    nc, ns = info.num_cores, info.num_subcores
    nw = nc * ns
    nbt = batch // BT
    n_blocks = L * nbt
    blocks_per_w = n_blocks // nw

    mesh = plsc.VectorSubcoreMesh(core_axis_name="c", subcore_axis_name="s")

    @functools.partial(
        pl.kernel,
        out_type=jax.ShapeDtypeStruct((L, D // 8, nbt, 8, BT), jnp.float32),
        mesh=mesh,
        scratch_types=[
            pltpu.VMEM((blocks_per_w, BT), jnp.int32),
            pltpu.VMEM((L, D), jnp.float32),
            pltpu.VMEM((BT, D), jnp.float32),
            pltpu.VMEM((BT, D), jnp.float32),
            pltpu.VMEM((D, TPAD), jnp.float32),
            pltpu.VMEM((D, TPAD), jnp.float32),
            pltpu.SemaphoreType.DMA,
            pltpu.SemaphoreType.DMA,
            pltpu.SemaphoreType.DMA,
            pltpu.SemaphoreType.DMA,
        ],
        compiler_params=pltpu.CompilerParams(
            use_tc_tiling_on_sc=False, needs_layout_passes=False
        ),
    )
    def k(x_hbm, table_hbm, pe_hbm, out_hbm,
          idx_all, pe_v, g0, g1, t0, t1, gs0, gs1, ws0, ws1):
        gbuf = (g0, g1)
        tbuf = (t0, t1)
        gsem = (gs0, gs1)
        wsem = (ws0, ws1)
        wid = lax.axis_index("s") * nc + lax.axis_index("c")
        blk_base = wid * blocks_per_w
        pltpu.sync_copy(pe_hbm, pe_v)
        pltpu.sync_copy(x_hbm.at[pl.ds(blk_base, blocks_per_w)], idx_all)

        iota = lax.iota(jnp.int32, LANES)
        rows = [iota + (g * LANES) for g in range(D // LANES)]

        def start_gather(kk, b):
            pltpu.async_copy(table_hbm.at[idx_all.at[kk]], gbuf[b], gsem[b])

        def wait_gather(kk, b):
            pltpu.make_async_copy(
                table_hbm.at[idx_all.at[kk]], gbuf[b], gsem[b]
            ).wait()

        def lc_of(kk):
            bid = blk_base + kk
            return (bid // (nbt * 8)) * 8 + bid % 8, (bid // 8) % nbt

        def start_writes(kk, b):
            l, tc = lc_of(kk)
            for tr in range(D // 8):
                pltpu.async_copy(
                    tbuf[b].at[pl.ds(tr * 8, 8), pl.ds(0, BT)],
                    out_hbm.at[l, tr, tc],
                    wsem[b],
                )

        def wait_writes(kk, b):
            l, tc = lc_of(kk)
            for tr in range(D // 8):
                pltpu.make_async_copy(
                    tbuf[b].at[pl.ds(tr * 8, 8), pl.ds(0, BT)],
                    out_hbm.at[l, tr, tc],
                    wsem[b],
                ).wait()

        for b in range(2):
            start_gather(b, b)

        @pl.loop(0, blocks_per_w, step=2)
        def _blocks(k2):
            for b in range(2):
                kk = k2 + b
                l, _ = lc_of(kk)
                wait_gather(kk, b)

                @pl.when(kk >= 2)
                def _():
                    wait_writes(kk - 2, b)

                pes = [pe_v[l, pl.ds(g * LANES, LANES)]
                       for g in range(D // LANES)]

                @plsc.parallel_loop(0, BT, unroll=4)
                def _tok(c):
                    col = jnp.broadcast_to(c, (LANES,))
                    for g in range(D // LANES):
                        v = gbuf[b][c, pl.ds(g * LANES, LANES)] + pes[g]
                        plsc.store_scatter(tbuf[b], [rows[g], col], v)

                @pl.when(kk + 2 < blocks_per_w)
                def _():
                    start_gather(kk + 2, b)

                start_writes(kk, b)

        for b in range(2):
            wait_writes(blocks_per_w - 2 + b, b)

    return k(xq, table, pe)


def kernel(x, table):
    batch, seq = x.shape
    vocab = table.shape[0]
    nbt = batch // BT
    xq = (x.astype(jnp.int32)
          .reshape(nbt, BT, seq // 8, 8)
          .transpose(2, 0, 3, 1)
          .reshape(seq * nbt, BT))
    pe = _pe_table()
    out5 = _run(xq, table, pe, batch=batch, vocab=vocab)
    return out5.transpose(2, 4, 0, 1, 3).reshape(batch, seq, D)


# speedup vs baseline: 13.4568x; 1.1115x over previous
import functools

import jax
import jax.numpy as jnp
from jax import lax
from jax.experimental import pallas as pl
from jax.experimental.pallas import tpu as pltpu
from jax.experimental.pallas import tpu_sc as plsc

L = 200
D = 64
LANES = 16
BT = 128
TPAD = 129
NBUF = 4


def _pe_table():
    position = jnp.arange(L, dtype=jnp.float32)[:, None]
    div_term = jnp.exp(
        jnp.arange(0, D, 2, dtype=jnp.float32) * (-jnp.log(10000.0) / D)
    )
    pe = jnp.zeros((L, D), dtype=jnp.float32)
    pe = pe.at[:, 0::2].set(jnp.sin(position * div_term))
    pe = pe.at[:, 1::2].set(jnp.cos(position * div_term))
    return pe


@functools.partial(jax.jit, static_argnames=("batch", "vocab"))
def _run(xq, table, pe, *, batch, vocab):
    info = plsc.get_sparse_core_info()
    nc, ns = info.num_cores, info.num_subcores
    nw = nc * ns
    nbt = batch // BT
    n_blocks = L * nbt
    blocks_per_w = n_blocks // nw

    mesh = plsc.VectorSubcoreMesh(core_axis_name="c", subcore_axis_name="s")

    @functools.partial(
        pl.kernel,
        out_type=jax.ShapeDtypeStruct((L, D // 8, nbt, 8, BT), jnp.float32),
        mesh=mesh,
        scratch_types=(
            [pltpu.VMEM((blocks_per_w, BT), jnp.int32),
             pltpu.VMEM((L, D), jnp.float32)]
            + [pltpu.VMEM((BT, D), jnp.float32)] * NBUF
            + [pltpu.VMEM((8, 8, TPAD), jnp.float32)] * NBUF
            + [pltpu.SemaphoreType.DMA] * (2 * NBUF)
        ),
        compiler_params=pltpu.CompilerParams(
            use_tc_tiling_on_sc=False, needs_layout_passes=False
        ),
    )
    def k(x_hbm, table_hbm, pe_hbm, out_hbm, idx_all, pe_v, *bufs):
        gbuf = bufs[:NBUF]
        tbuf = bufs[NBUF:2 * NBUF]
        gsem = bufs[2 * NBUF:3 * NBUF]
        wsem = bufs[3 * NBUF:]
        wid = lax.axis_index("s") * nc + lax.axis_index("c")
        blk_base = wid * blocks_per_w
        pltpu.sync_copy(pe_hbm, pe_v)
        pltpu.sync_copy(x_hbm.at[pl.ds(blk_base, blocks_per_w)], idx_all)

        iota = lax.iota(jnp.int32, LANES)
        trs = [iota // 8 + (g * 2) for g in range(D // LANES)]
        rrs = [iota % 8 for g in range(D // LANES)]

        def start_gather(kk, b):
            pltpu.async_copy(table_hbm.at[idx_all.at[kk]], gbuf[b], gsem[b])

        def wait_gather(kk, b):
            pltpu.make_async_copy(
                table_hbm.at[idx_all.at[kk]], gbuf[b], gsem[b]
            ).wait()

        def lc_of(kk):
            bid = blk_base + kk
            return (bid // (nbt * 8)) * 8 + bid % 8, (bid // 8) % nbt

        def start_write(kk, b):
            l, tc = lc_of(kk)
            pltpu.async_copy(
                tbuf[b].at[pl.ds(0, 8), pl.ds(0, 8), pl.ds(0, BT)],
                out_hbm.at[l, :, tc],
                wsem[b],
            )

        def wait_write(kk, b):
            l, tc = lc_of(kk)
            pltpu.make_async_copy(
                tbuf[b].at[pl.ds(0, 8), pl.ds(0, 8), pl.ds(0, BT)],
                out_hbm.at[l, :, tc],
                wsem[b],
            ).wait()

        for b in range(NBUF):
            start_gather(b, b)

        @pl.loop(0, blocks_per_w, step=NBUF)
        def _blocks(k2):
            for b in range(NBUF):
                kk = k2 + b
                l, _ = lc_of(kk)
                wait_gather(kk, b)

                @pl.when(kk >= NBUF)
                def _():
                    wait_write(kk - NBUF, b)

                pes = [pe_v[l, pl.ds(g * LANES, LANES)]
                       for g in range(D // LANES)]

                @plsc.parallel_loop(0, BT, unroll=4)
                def _tok(c):
                    col = jnp.broadcast_to(c, (LANES,))
                    for g in range(D // LANES):
                        v = gbuf[b][c, pl.ds(g * LANES, LANES)] + pes[g]
                        plsc.store_scatter(tbuf[b], [trs[g], rrs[g], col], v)

                @pl.when(kk + NBUF < blocks_per_w)
                def _():
                    start_gather(kk + NBUF, b)

                start_write(kk, b)

        for b in range(NBUF):
            wait_write(blocks_per_w - NBUF + b, b)

    return k(xq, table, pe)


def kernel(x, table):
    batch, seq = x.shape
    vocab = table.shape[0]
    nbt = batch // BT
    xq = (x.astype(jnp.int32)
          .reshape(nbt, BT, seq // 8, 8)
          .transpose(2, 0, 3, 1)
          .reshape(seq * nbt, BT))
    pe = _pe_table()
    out5 = _run(xq, table, pe, batch=batch, vocab=vocab)
    return out5.transpose(2, 4, 0, 1, 3).reshape(batch, seq, D)
